# baseline (speedup 1.0000x reference)
import functools

import jax
import jax.numpy as jnp
from jax import lax
from jax.experimental import pallas as pl
from jax.experimental.pallas import tpu as pltpu
from jax.experimental.pallas import tpu_sc as plsc

B = 16
L = 4096
D = 300
RB = 64
CHUNKS = (D + 15) // 16
TAIL_LIVE = D - (CHUNKS - 1) * 16
ACC_W = CHUNKS * 16
BLK = RB * D
NTILES = 32

_mesh = plsc.VectorSubcoreMesh(core_axis_name="c", subcore_axis_name="s")


@functools.partial(
    pl.kernel,
    out_type=jax.ShapeDtypeStruct((NTILES * B * ACC_W,), jnp.float32),
    mesh=_mesh,
    compiler_params=pltpu.CompilerParams(needs_layout_passes=False),
    scratch_types=[
        pltpu.VMEM((16,), jnp.int32),
        pltpu.VMEM((BLK + 16,), jnp.float32),
        pltpu.VMEM((B * ACC_W,), jnp.float32),
    ],
)
def _sc_partial_sums(x_ref, len_ref, out_ref, len_v, buf, acc):
    wid = lax.axis_index("s") * 2 + lax.axis_index("c")

    zero = jnp.zeros((16,), jnp.float32)

    def _zero_acc(i, carry):
        acc[pl.ds(i * 16, 16)] = zero
        return carry

    lax.fori_loop(0, B * ACC_W // 16, _zero_acc, 0)

    pltpu.sync_copy(len_ref, len_v)
    lv = len_v[...]
    nb = (lv + (RB - 1)) // RB
    cum = plsc.cumsum(nb)
    total = jnp.sum(nb)
    idx16 = lax.broadcasted_iota(jnp.int32, (16,), 0)
    tail_mask = idx16 < TAIL_LIVE

    def block_body(g):
        before = cum <= g
        b = jnp.sum(jnp.where(before, 1, 0))
        excl_b = jnp.sum(jnp.where(before, nb, 0))
        len_b = jnp.sum(jnp.where(idx16 == b, lv, 0))
        local = g - excl_b
        row0 = b * L + local * RB
        nrows = jnp.minimum(RB, len_b - local * RB)
        pltpu.sync_copy(x_ref.at[pl.ds(row0 * D, BLK)], buf.at[pl.ds(0, BLK)])

        def row_body(r, accs):
            base = r * D
            new = []
            for j in range(CHUNKS):
                v = buf[pl.ds(base + j * 16, 16)]
                if j == CHUNKS - 1:
                    v = jnp.where(tail_mask, v, 0.0)
                new.append(accs[j] + v)
            return tuple(new)

        accs = lax.fori_loop(0, nrows, row_body,
                             tuple(zero for _ in range(CHUNKS)))
        rowoff = b * ACC_W
        for j in range(CHUNKS):
            sl = pl.ds(rowoff + j * 16, 16)
            acc[sl] = acc[sl] + accs[j]
        return g + NTILES

    lax.while_loop(lambda g: g < total, block_body, wid)

    pltpu.sync_copy(acc, out_ref.at[pl.ds(wid * B * ACC_W, B * ACC_W)])


def _tc_finish(p_ref, len_ref, o_ref):
    s = jnp.sum(p_ref[...], axis=0)
    o_ref[...] = s[:, :D] / len_ref[...]


def kernel(sentences, sentence_lengths):
    x = sentences.reshape(-1)
    partials = _sc_partial_sums(x, sentence_lengths)
    p3 = partials.reshape(NTILES, B, ACC_W)
    lf = sentence_lengths.astype(jnp.float32).reshape(B, 1)
    return pl.pallas_call(
        _tc_finish,
        out_shape=jax.ShapeDtypeStruct((B, D), jnp.float32),
    )(p3, lf)


# speedup vs baseline: 1.3378x; 1.3378x over previous
import functools

import jax
import jax.numpy as jnp
from jax import lax
from jax.experimental import pallas as pl
from jax.experimental.pallas import tpu as pltpu
from jax.experimental.pallas import tpu_sc as plsc

B = 16
L = 4096
D = 300
RB = 64
CHUNKS = (D + 15) // 16
TAIL_LIVE = D - (CHUNKS - 1) * 16
ACC_W = CHUNKS * 16
BLK = RB * D
NTILES = 32

BLKP = BLK + 16


@functools.cache
def _make_sc_partial_sums():
    mesh = plsc.VectorSubcoreMesh(core_axis_name="c", subcore_axis_name="s")
    return functools.partial(
        pl.kernel,
        out_type=jax.ShapeDtypeStruct((NTILES * B * ACC_W,), jnp.float32),
        mesh=mesh,
        compiler_params=pltpu.CompilerParams(needs_layout_passes=False),
        scratch_types=[
            pltpu.VMEM((16,), jnp.int32),
            pltpu.VMEM((16,), jnp.int32),
            pltpu.VMEM((2 * BLKP,), jnp.float32),
            pltpu.VMEM((B * ACC_W,), jnp.float32),
            pltpu.SemaphoreType.DMA,
            pltpu.SemaphoreType.DMA,
        ],
    )(_sc_partial_sums_body)


def _sc_partial_sums_body(x_ref, len_ref, start_ref, out_ref,
                          len_v, start_v, buf, acc, sem0, sem1):
    wid = lax.axis_index("s") * 2 + lax.axis_index("c")

    zero = jnp.zeros((16,), jnp.float32)

    def _zero_acc(i, carry):
        acc[pl.ds(i * 16, 16)] = zero
        return carry

    lax.fori_loop(0, B * ACC_W // 16, _zero_acc, 0)

    pltpu.sync_copy(len_ref, len_v)
    pltpu.sync_copy(start_ref, start_v)
    lv = len_v[...]
    sv = start_v[...]
    seg = lv - sv
    nb = (seg + (RB - 1)) // RB
    cum = plsc.cumsum(nb)
    total = jnp.sum(nb)
    idx16 = lax.broadcasted_iota(jnp.int32, (16,), 0)
    tail_mask = idx16 < TAIL_LIVE

    def block_info(g):
        before = cum <= g
        b = jnp.sum(jnp.where(before, 1, 0))
        excl_b = jnp.sum(jnp.where(before, nb, 0))
        is_b = idx16 == b
        seg_b = jnp.sum(jnp.where(is_b, seg, 0))
        start_b = jnp.sum(jnp.where(is_b, sv, 0))
        local = g - excl_b
        row0 = b * L + start_b + local * RB
        nrows = jnp.minimum(RB, seg_b - local * RB)
        return b, row0, nrows

    def copy_op(g, p, sem):
        _, row0, _ = block_info(g)
        off = pl.multiple_of(row0 * D, 8)
        return pltpu.make_async_copy(
            x_ref.at[pl.ds(off, BLK)],
            buf.at[pl.ds(p * BLKP, BLK)],
            sem,
        )

    @pl.when(wid < total)
    def _prime():
        copy_op(wid, 0, sem0).start()

    def block_body(k):
        g = wid + k * NTILES
        p = lax.rem(k, 2)
        gn = g + NTILES

        @pl.when(jnp.logical_and(gn < total, p == 0))
        def _issue_next0():
            copy_op(gn, 1, sem1).start()

        @pl.when(jnp.logical_and(gn < total, p == 1))
        def _issue_next1():
            copy_op(gn, 0, sem0).start()

        @pl.when(p == 0)
        def _wait0():
            copy_op(g, 0, sem0).wait()

        @pl.when(p == 1)
        def _wait1():
            copy_op(g, 1, sem1).wait()

        b, _, nrows = block_info(g)
        boff = p * BLKP

        def row_body(r, accs):
            base = boff + r * D
            new = []
            for j in range(CHUNKS):
                v = buf[pl.ds(base + j * 16, 16)]
                if j == CHUNKS - 1:
                    v = jnp.where(tail_mask, v, 0.0)
                new.append(accs[j] + v)
            return tuple(new)

        accs = lax.fori_loop(0, nrows, row_body,
                             tuple(zero for _ in range(CHUNKS)))
        rowoff = b * ACC_W
        for j in range(CHUNKS):
            sl = pl.ds(rowoff + j * 16, 16)
            acc[sl] = acc[sl] + accs[j]
        return k + 1

    nblocks_mine = lax.div(total - wid + NTILES - 1, NTILES)

    def cond(k):
        return k < nblocks_mine

    lax.while_loop(cond, block_body, 0)

    pltpu.sync_copy(acc, out_ref.at[pl.ds(wid * B * ACC_W, B * ACC_W)])


RB_TC = 256
NL_TC = L // RB_TC


def _tc_ragged_body(len_ref, x_ref, o_ref):
    b = pl.program_id(0)
    l = pl.program_id(1)
    len_b = len_ref[b]
    nlive = (len_b + RB_TC - 1) // RB_TC
    x = x_ref[0]
    start = l * RB_TC
    rows = lax.broadcasted_iota(jnp.int32, (RB_TC, 1), 0) + start
    s = jnp.sum(jnp.where(rows < len_b, x, 0.0), axis=0)[None, None]

    @pl.when(l == 0)
    def _init():
        o_ref[...] = jnp.zeros_like(o_ref)

    @pl.when(l < nlive)
    def _acc():
        o_ref[...] = o_ref[...] + s

    @pl.when(l == NL_TC - 1)
    def _fin():
        o_ref[...] = o_ref[...] / len_b.astype(jnp.float32)


def _tc_ragged(sentences, sentence_lengths):
    grid_spec = pltpu.PrefetchScalarGridSpec(
        num_scalar_prefetch=1,
        grid=(B, NL_TC),
        in_specs=[
            pl.BlockSpec(
                (1, RB_TC, D),
                lambda b, l, lens: (b, jnp.minimum(l, (lens[b] + RB_TC - 1) // RB_TC - 1), 0),
            ),
        ],
        out_specs=pl.BlockSpec((1, 1, D), lambda b, l, lens: (b, 0, 0)),
    )
    return pl.pallas_call(
        _tc_ragged_body,
        grid_spec=grid_spec,
        out_shape=jax.ShapeDtypeStruct((B, 1, D), jnp.float32),
        compiler_params=pltpu.CompilerParams(
            dimension_semantics=("arbitrary", "arbitrary"),
        ),
    )(sentence_lengths, sentences).reshape(B, D)


def _tc_finish(p_ref, len_ref, o_ref):
    s = jnp.sum(p_ref[...], axis=0)
    o_ref[...] = s[:, :D] / len_ref[...]


def _tc_sum_body(len_ref, x_ref, o_ref):
    b = pl.program_id(0)
    l = pl.program_id(1)
    len_b = len_ref[b]
    nlive = (len_b + RB_TC - 1) // RB_TC
    x = x_ref[0]
    rows = lax.broadcasted_iota(jnp.int32, (RB_TC, 1), 0) + l * RB_TC
    s = jnp.sum(jnp.where(rows < len_b, x, 0.0), axis=0)[None, None]

    @pl.when(l == 0)
    def _init():
        o_ref[...] = jnp.zeros_like(o_ref)

    @pl.when(l < nlive)
    def _acc():
        o_ref[...] = o_ref[...] + s


def _tc_sum(sentences, tc_lens):
    grid_spec = pltpu.PrefetchScalarGridSpec(
        num_scalar_prefetch=1,
        grid=(B, NL_TC),
        in_specs=[
            pl.BlockSpec(
                (1, RB_TC, D),
                lambda b, l, lens: (
                    b,
                    jnp.maximum(
                        jnp.minimum(l, (lens[b] + RB_TC - 1) // RB_TC - 1), 0
                    ),
                    0,
                ),
            ),
        ],
        out_specs=pl.BlockSpec((1, 1, D), lambda b, l, lens: (b, 0, 0)),
    )
    return pl.pallas_call(
        _tc_sum_body,
        grid_spec=grid_spec,
        out_shape=jax.ShapeDtypeStruct((B, 1, D), jnp.float32),
        compiler_params=pltpu.CompilerParams(
            dimension_semantics=("arbitrary", "arbitrary"),
        ),
    )(tc_lens, sentences)


def _combine_finish(p_ref, t_ref, len_ref, o_ref):
    s = jnp.sum(p_ref[...], axis=0)
    o_ref[...] = (s[:, :D] + t_ref[...]) / len_ref[...]


FTC_NUM = 5
FTC_DEN = 8


def _combined(sentences, sentence_lengths):
    tc_lens = (sentence_lengths * FTC_NUM // FTC_DEN) // RB * RB
    x = sentences.reshape(-1)
    t = _tc_sum(sentences, tc_lens)
    p = _make_sc_partial_sums()(x, sentence_lengths, tc_lens)
    return pl.pallas_call(
        _combine_finish,
        out_shape=jax.ShapeDtypeStruct((B, D), jnp.float32),
    )(
        p.reshape(NTILES, B, ACC_W),
        t.reshape(B, D),
        sentence_lengths.astype(jnp.float32).reshape(B, 1),
    )


def kernel(sentences, sentence_lengths):
    return _tc_ragged(sentences, sentence_lengths)


# speedup vs baseline: 1.8605x; 1.3907x over previous
import functools

import jax
import jax.numpy as jnp
from jax import lax
from jax.experimental import pallas as pl
from jax.experimental.pallas import tpu as pltpu
from jax.experimental.pallas import tpu_sc as plsc

B = 16
L = 4096
D = 300
RB = 64
CHUNKS = (D + 15) // 16
TAIL_LIVE = D - (CHUNKS - 1) * 16
ACC_W = CHUNKS * 16
BLK = RB * D
NTILES = 32

BLKP = BLK + 16


@functools.cache
def _make_sc_partial_sums():
    mesh = plsc.VectorSubcoreMesh(core_axis_name="c", subcore_axis_name="s")
    return functools.partial(
        pl.kernel,
        out_type=jax.ShapeDtypeStruct((NTILES * B * ACC_W,), jnp.float32),
        mesh=mesh,
        compiler_params=pltpu.CompilerParams(needs_layout_passes=False),
        scratch_types=[
            pltpu.VMEM((16,), jnp.int32),
            pltpu.VMEM((16,), jnp.int32),
            pltpu.VMEM((2 * BLKP,), jnp.float32),
            pltpu.VMEM((B * ACC_W,), jnp.float32),
            pltpu.SemaphoreType.DMA,
            pltpu.SemaphoreType.DMA,
        ],
    )(_sc_partial_sums_body)


def _sc_partial_sums_body(x_ref, len_ref, start_ref, out_ref,
                          len_v, start_v, buf, acc, sem0, sem1):
    wid = lax.axis_index("s") * 2 + lax.axis_index("c")

    zero = jnp.zeros((16,), jnp.float32)

    def _zero_acc(i, carry):
        acc[pl.ds(i * 16, 16)] = zero
        return carry

    lax.fori_loop(0, B * ACC_W // 16, _zero_acc, 0)

    pltpu.sync_copy(len_ref, len_v)
    pltpu.sync_copy(start_ref, start_v)
    lv = len_v[...]
    sv = start_v[...]
    seg = lv - sv
    nb = (seg + (RB - 1)) // RB
    cum = plsc.cumsum(nb)
    total = jnp.sum(nb)
    idx16 = lax.broadcasted_iota(jnp.int32, (16,), 0)
    tail_mask = idx16 < TAIL_LIVE

    def block_info(g):
        before = cum <= g
        b = jnp.sum(jnp.where(before, 1, 0))
        excl_b = jnp.sum(jnp.where(before, nb, 0))
        is_b = idx16 == b
        seg_b = jnp.sum(jnp.where(is_b, seg, 0))
        start_b = jnp.sum(jnp.where(is_b, sv, 0))
        local = g - excl_b
        row0 = b * L + start_b + local * RB
        nrows = jnp.minimum(RB, seg_b - local * RB)
        return b, row0, nrows

    def copy_op(g, p, sem):
        _, row0, _ = block_info(g)
        off = pl.multiple_of(row0 * D, 8)
        return pltpu.make_async_copy(
            x_ref.at[pl.ds(off, BLK)],
            buf.at[pl.ds(p * BLKP, BLK)],
            sem,
        )

    @pl.when(wid < total)
    def _prime():
        copy_op(wid, 0, sem0).start()

    def block_body(k):
        g = wid + k * NTILES
        p = lax.rem(k, 2)
        gn = g + NTILES

        @pl.when(jnp.logical_and(gn < total, p == 0))
        def _issue_next0():
            copy_op(gn, 1, sem1).start()

        @pl.when(jnp.logical_and(gn < total, p == 1))
        def _issue_next1():
            copy_op(gn, 0, sem0).start()

        @pl.when(p == 0)
        def _wait0():
            copy_op(g, 0, sem0).wait()

        @pl.when(p == 1)
        def _wait1():
            copy_op(g, 1, sem1).wait()

        b, _, nrows = block_info(g)
        boff = p * BLKP

        def row_body(r, accs):
            base = boff + r * D
            new = []
            for j in range(CHUNKS):
                v = buf[pl.ds(base + j * 16, 16)]
                if j == CHUNKS - 1:
                    v = jnp.where(tail_mask, v, 0.0)
                new.append(accs[j] + v)
            return tuple(new)

        accs = lax.fori_loop(0, nrows, row_body,
                             tuple(zero for _ in range(CHUNKS)))
        rowoff = b * ACC_W
        for j in range(CHUNKS):
            sl = pl.ds(rowoff + j * 16, 16)
            acc[sl] = acc[sl] + accs[j]
        return k + 1

    nblocks_mine = lax.div(total - wid + NTILES - 1, NTILES)

    def cond(k):
        return k < nblocks_mine

    lax.while_loop(cond, block_body, 0)

    pltpu.sync_copy(acc, out_ref.at[pl.ds(wid * B * ACC_W, B * ACC_W)])


RB_TC = 256
NL_TC = L // RB_TC


def _tc_ragged_body(len_ref, x_ref, o_ref):
    b = pl.program_id(0)
    l = pl.program_id(1)
    len_b = len_ref[b]
    nlive = (len_b + RB_TC - 1) // RB_TC
    x = x_ref[0]
    start = l * RB_TC
    rows = lax.broadcasted_iota(jnp.int32, (RB_TC, 1), 0) + start
    s = jnp.sum(jnp.where(rows < len_b, x, 0.0), axis=0)[None, None]

    @pl.when(l == 0)
    def _init():
        o_ref[...] = jnp.zeros_like(o_ref)

    @pl.when(l < nlive)
    def _acc():
        o_ref[...] = o_ref[...] + s

    @pl.when(l == NL_TC - 1)
    def _fin():
        o_ref[...] = o_ref[...] / len_b.astype(jnp.float32)


def _tc_ragged(sentences, sentence_lengths):
    grid_spec = pltpu.PrefetchScalarGridSpec(
        num_scalar_prefetch=1,
        grid=(B, NL_TC),
        in_specs=[
            pl.BlockSpec(
                (1, RB_TC, D),
                lambda b, l, lens: (b, jnp.minimum(l, (lens[b] + RB_TC - 1) // RB_TC - 1), 0),
            ),
        ],
        out_specs=pl.BlockSpec((1, 1, D), lambda b, l, lens: (b, 0, 0)),
    )
    return pl.pallas_call(
        _tc_ragged_body,
        grid_spec=grid_spec,
        out_shape=jax.ShapeDtypeStruct((B, 1, D), jnp.float32),
        compiler_params=pltpu.CompilerParams(
            dimension_semantics=("arbitrary", "arbitrary"),
        ),
    )(sentence_lengths, sentences).reshape(B, D)


DEPTH = 4
RBM = 256


def _tc_manual_body(len_ref, x_ref, o_ref, ring, acc, sems, *, divide):
    b = pl.program_id(0)
    len_b = len_ref[b]
    nlive = (len_b + RBM - 1) // RBM
    base = b * L

    def copy_op(k, slot):
        return pltpu.make_async_copy(
            x_ref.at[pl.ds(base + k * RBM, RBM), :],
            ring.at[slot],
            sems.at[slot],
        )

    for d in range(DEPTH):
        @pl.when(d < nlive)
        def _prime():
            copy_op(d, d).start()

    def step(k, carry):
        slot = lax.rem(k, DEPTH)
        copy_op(k, slot).wait()
        v = ring[slot]
        rows = lax.broadcasted_iota(jnp.int32, (RBM, 1), 0) + k * RBM
        v = jnp.where(rows < len_b, v, 0.0)

        @pl.when(k == 0)
        def _first():
            acc[...] = v

        @pl.when(k > 0)
        def _rest():
            acc[...] = acc[...] + v

        @pl.when(k + DEPTH < nlive)
        def _refill():
            copy_op(k + DEPTH, slot).start()

        return carry

    lax.fori_loop(0, nlive, step, 0)

    s = jnp.sum(acc[...], axis=0)[None, None]
    s = jnp.where(nlive > 0, s, 0.0)
    if divide:
        s = s / len_b.astype(jnp.float32)
    o_ref[...] = s


def _tc_manual(sentences, lens, divide):
    x2d = sentences.reshape(B * L, D)
    grid_spec = pltpu.PrefetchScalarGridSpec(
        num_scalar_prefetch=1,
        grid=(B,),
        in_specs=[pl.BlockSpec(memory_space=pl.ANY)],
        out_specs=pl.BlockSpec((1, 1, D), lambda b, lens: (b, 0, 0)),
        scratch_shapes=[
            pltpu.VMEM((DEPTH, RBM, D), jnp.float32),
            pltpu.VMEM((RBM, D), jnp.float32),
            pltpu.SemaphoreType.DMA((DEPTH,)),
        ],
    )
    return pl.pallas_call(
        functools.partial(_tc_manual_body, divide=divide),
        grid_spec=grid_spec,
        out_shape=jax.ShapeDtypeStruct((B, 1, D), jnp.float32),
        compiler_params=pltpu.CompilerParams(
            dimension_semantics=("arbitrary",),
        ),
    )(lens, x2d)


def _tc_finish(p_ref, len_ref, o_ref):
    s = jnp.sum(p_ref[...], axis=0)
    o_ref[...] = s[:, :D] / len_ref[...]


def _tc_sum_body(len_ref, x_ref, o_ref):
    b = pl.program_id(0)
    l = pl.program_id(1)
    len_b = len_ref[b]
    nlive = (len_b + RB_TC - 1) // RB_TC
    x = x_ref[0]
    rows = lax.broadcasted_iota(jnp.int32, (RB_TC, 1), 0) + l * RB_TC
    s = jnp.sum(jnp.where(rows < len_b, x, 0.0), axis=0)[None, None]

    @pl.when(l == 0)
    def _init():
        o_ref[...] = jnp.zeros_like(o_ref)

    @pl.when(l < nlive)
    def _acc():
        o_ref[...] = o_ref[...] + s


def _tc_sum(sentences, tc_lens):
    grid_spec = pltpu.PrefetchScalarGridSpec(
        num_scalar_prefetch=1,
        grid=(B, NL_TC),
        in_specs=[
            pl.BlockSpec(
                (1, RB_TC, D),
                lambda b, l, lens: (
                    b,
                    jnp.maximum(
                        jnp.minimum(l, (lens[b] + RB_TC - 1) // RB_TC - 1), 0
                    ),
                    0,
                ),
            ),
        ],
        out_specs=pl.BlockSpec((1, 1, D), lambda b, l, lens: (b, 0, 0)),
    )
    return pl.pallas_call(
        _tc_sum_body,
        grid_spec=grid_spec,
        out_shape=jax.ShapeDtypeStruct((B, 1, D), jnp.float32),
        compiler_params=pltpu.CompilerParams(
            dimension_semantics=("arbitrary", "arbitrary"),
        ),
    )(tc_lens, sentences)


def _combine_finish(p_ref, t_ref, len_ref, o_ref):
    s = jnp.sum(p_ref[...], axis=0)
    o_ref[...] = (s[:, :D] + t_ref[...]) / len_ref[...]


FTC_NUM = 5
FTC_DEN = 8


def _combined(sentences, sentence_lengths):
    tc_lens = (sentence_lengths * FTC_NUM // FTC_DEN) // RB * RB
    x = sentences.reshape(-1)
    t = _tc_sum(sentences, tc_lens)
    p = _make_sc_partial_sums()(x, sentence_lengths, tc_lens)
    return pl.pallas_call(
        _combine_finish,
        out_shape=jax.ShapeDtypeStruct((B, D), jnp.float32),
    )(
        p.reshape(NTILES, B, ACC_W),
        t.reshape(B, D),
        sentence_lengths.astype(jnp.float32).reshape(B, 1),
    )


def kernel(sentences, sentence_lengths):
    return _tc_manual(sentences, sentence_lengths, divide=True).reshape(B, D)


# speedup vs baseline: 7.9042x; 4.2484x over previous
import functools

import jax
import jax.numpy as jnp
from jax import lax
from jax.experimental import pallas as pl
from jax.experimental.pallas import tpu as pltpu
from jax.experimental.pallas import tpu_sc as plsc

B = 16
L = 4096
D = 300
RB = 64
CHUNKS = (D + 15) // 16
TAIL_LIVE = D - (CHUNKS - 1) * 16
ACC_W = CHUNKS * 16
BLK = RB * D
NTILES = 32

BLKP = BLK + 16


@functools.cache
def _make_sc_partial_sums():
    mesh = plsc.VectorSubcoreMesh(core_axis_name="c", subcore_axis_name="s")
    return functools.partial(
        pl.kernel,
        out_type=jax.ShapeDtypeStruct((NTILES * B * ACC_W,), jnp.float32),
        mesh=mesh,
        compiler_params=pltpu.CompilerParams(needs_layout_passes=False),
        scratch_types=[
            pltpu.VMEM((16,), jnp.int32),
            pltpu.VMEM((16,), jnp.int32),
            pltpu.VMEM((2 * BLKP,), jnp.float32),
            pltpu.VMEM((B * ACC_W,), jnp.float32),
            pltpu.SemaphoreType.DMA,
            pltpu.SemaphoreType.DMA,
        ],
    )(_sc_partial_sums_body)


def _sc_partial_sums_body(x_ref, len_ref, start_ref, out_ref,
                          len_v, start_v, buf, acc, sem0, sem1):
    wid = lax.axis_index("s") * 2 + lax.axis_index("c")

    zero = jnp.zeros((16,), jnp.float32)

    def _zero_acc(i, carry):
        acc[pl.ds(i * 16, 16)] = zero
        return carry

    lax.fori_loop(0, B * ACC_W // 16, _zero_acc, 0)

    pltpu.sync_copy(len_ref, len_v)
    pltpu.sync_copy(start_ref, start_v)
    lv = len_v[...]
    sv = start_v[...]
    seg = lv - sv
    nb = (seg + (RB - 1)) // RB
    cum = plsc.cumsum(nb)
    total = jnp.sum(nb)
    idx16 = lax.broadcasted_iota(jnp.int32, (16,), 0)
    tail_mask = idx16 < TAIL_LIVE

    def block_info(g):
        before = cum <= g
        b = jnp.sum(jnp.where(before, 1, 0))
        excl_b = jnp.sum(jnp.where(before, nb, 0))
        is_b = idx16 == b
        seg_b = jnp.sum(jnp.where(is_b, seg, 0))
        start_b = jnp.sum(jnp.where(is_b, sv, 0))
        local = g - excl_b
        row0 = b * L + start_b + local * RB
        nrows = jnp.minimum(RB, seg_b - local * RB)
        return b, row0, nrows

    def copy_op(g, p, sem):
        _, row0, _ = block_info(g)
        off = pl.multiple_of(row0 * D, 8)
        return pltpu.make_async_copy(
            x_ref.at[pl.ds(off, BLK)],
            buf.at[pl.ds(p * BLKP, BLK)],
            sem,
        )

    @pl.when(wid < total)
    def _prime():
        copy_op(wid, 0, sem0).start()

    def block_body(k):
        g = wid + k * NTILES
        p = lax.rem(k, 2)
        gn = g + NTILES

        @pl.when(jnp.logical_and(gn < total, p == 0))
        def _issue_next0():
            copy_op(gn, 1, sem1).start()

        @pl.when(jnp.logical_and(gn < total, p == 1))
        def _issue_next1():
            copy_op(gn, 0, sem0).start()

        @pl.when(p == 0)
        def _wait0():
            copy_op(g, 0, sem0).wait()

        @pl.when(p == 1)
        def _wait1():
            copy_op(g, 1, sem1).wait()

        b, _, nrows = block_info(g)
        boff = p * BLKP

        def row_body(r, accs):
            base = boff + r * D
            new = []
            for j in range(CHUNKS):
                v = buf[pl.ds(base + j * 16, 16)]
                if j == CHUNKS - 1:
                    v = jnp.where(tail_mask, v, 0.0)
                new.append(accs[j] + v)
            return tuple(new)

        accs = lax.fori_loop(0, nrows, row_body,
                             tuple(zero for _ in range(CHUNKS)))
        rowoff = b * ACC_W
        for j in range(CHUNKS):
            sl = pl.ds(rowoff + j * 16, 16)
            acc[sl] = acc[sl] + accs[j]
        return k + 1

    nblocks_mine = lax.div(total - wid + NTILES - 1, NTILES)

    def cond(k):
        return k < nblocks_mine

    lax.while_loop(cond, block_body, 0)

    pltpu.sync_copy(acc, out_ref.at[pl.ds(wid * B * ACC_W, B * ACC_W)])


RB_TC = 256
NL_TC = L // RB_TC


def _tc_ragged_body(len_ref, x_ref, o_ref):
    b = pl.program_id(0)
    l = pl.program_id(1)
    len_b = len_ref[b]
    nlive = (len_b + RB_TC - 1) // RB_TC
    x = x_ref[0]
    start = l * RB_TC
    rows = lax.broadcasted_iota(jnp.int32, (RB_TC, 1), 0) + start
    s = jnp.sum(jnp.where(rows < len_b, x, 0.0), axis=0)[None, None]

    @pl.when(l == 0)
    def _init():
        o_ref[...] = jnp.zeros_like(o_ref)

    @pl.when(l < nlive)
    def _acc():
        o_ref[...] = o_ref[...] + s

    @pl.when(l == NL_TC - 1)
    def _fin():
        o_ref[...] = o_ref[...] / len_b.astype(jnp.float32)


def _tc_ragged(sentences, sentence_lengths):
    grid_spec = pltpu.PrefetchScalarGridSpec(
        num_scalar_prefetch=1,
        grid=(B, NL_TC),
        in_specs=[
            pl.BlockSpec(
                (1, RB_TC, D),
                lambda b, l, lens: (b, jnp.minimum(l, (lens[b] + RB_TC - 1) // RB_TC - 1), 0),
            ),
        ],
        out_specs=pl.BlockSpec((1, 1, D), lambda b, l, lens: (b, 0, 0)),
    )
    return pl.pallas_call(
        _tc_ragged_body,
        grid_spec=grid_spec,
        out_shape=jax.ShapeDtypeStruct((B, 1, D), jnp.float32),
        compiler_params=pltpu.CompilerParams(
            dimension_semantics=("arbitrary", "arbitrary"),
        ),
    )(sentence_lengths, sentences).reshape(B, D)


DEPTH = 4
RBM = 256


def _tc_manual_body(len_ref, x_ref, o_ref, ring, acc, sems, *, divide):
    b = pl.program_id(0)
    len_b = len_ref[b]
    nlive = (len_b + RBM - 1) // RBM
    base = b * L

    def copy_op(k, slot):
        return pltpu.make_async_copy(
            x_ref.at[pl.ds(base + k * RBM, RBM), :],
            ring.at[slot],
            sems.at[slot],
        )

    for d in range(DEPTH):
        @pl.when(d < nlive)
        def _prime():
            copy_op(d, d).start()

    def step(k, carry):
        slot = lax.rem(k, DEPTH)
        copy_op(k, slot).wait()
        v = ring[slot]
        rows = lax.broadcasted_iota(jnp.int32, (RBM, 1), 0) + k * RBM
        v = jnp.where(rows < len_b, v, 0.0)

        @pl.when(k == 0)
        def _first():
            acc[...] = v

        @pl.when(k > 0)
        def _rest():
            acc[...] = acc[...] + v

        @pl.when(k + DEPTH < nlive)
        def _refill():
            copy_op(k + DEPTH, slot).start()

        return carry

    lax.fori_loop(0, nlive, step, 0)

    s = jnp.sum(acc[...], axis=0)[None, None]
    s = jnp.where(nlive > 0, s, 0.0)
    if divide:
        s = s / len_b.astype(jnp.float32)
    o_ref[...] = s


def _tc_manual(sentences, lens, divide):
    x2d = sentences.reshape(B * L, D)
    grid_spec = pltpu.PrefetchScalarGridSpec(
        num_scalar_prefetch=1,
        grid=(B,),
        in_specs=[pl.BlockSpec(memory_space=pl.ANY)],
        out_specs=pl.BlockSpec((1, 1, D), lambda b, lens: (b, 0, 0)),
        scratch_shapes=[
            pltpu.VMEM((DEPTH, RBM, D), jnp.float32),
            pltpu.VMEM((RBM, D), jnp.float32),
            pltpu.SemaphoreType.DMA((DEPTH,)),
        ],
    )
    return pl.pallas_call(
        functools.partial(_tc_manual_body, divide=divide),
        grid_spec=grid_spec,
        out_shape=jax.ShapeDtypeStruct((B, 1, D), jnp.float32),
        compiler_params=pltpu.CompilerParams(
            dimension_semantics=("arbitrary",),
        ),
    )(lens, x2d)


DBLK = 25


def _tc_native_body(x_ref, li_ref, lf_ref, o_ref):
    i = pl.program_id(0)
    x = x_ref[...]
    iota_l = lax.broadcasted_iota(jnp.int32, (B, L), 1)
    mask = iota_l < li_ref[...]
    s = jnp.sum(jnp.where(mask[None], x, 0.0), axis=2)
    o_ref[pl.ds(i * DBLK, DBLK), :] = s / lf_ref[...]


def _tc_native(sentences, sentence_lengths):
    x_t = jnp.transpose(sentences, (2, 0, 1))
    li = sentence_lengths.reshape(B, 1)
    lf = sentence_lengths.astype(jnp.float32).reshape(1, B)
    out = pl.pallas_call(
        _tc_native_body,
        grid=(D // DBLK,),
        in_specs=[
            pl.BlockSpec((DBLK, B, L), lambda i: (i, 0, 0)),
            pl.BlockSpec((B, 1), lambda i: (0, 0)),
            pl.BlockSpec((1, B), lambda i: (0, 0)),
        ],
        out_specs=pl.BlockSpec((D, B), lambda i: (0, 0)),
        out_shape=jax.ShapeDtypeStruct((D, B), jnp.float32),
        compiler_params=pltpu.CompilerParams(
            dimension_semantics=("arbitrary",),
        ),
    )(x_t, li, lf)
    return out.T


def _tc_finish(p_ref, len_ref, o_ref):
    s = jnp.sum(p_ref[...], axis=0)
    o_ref[...] = s[:, :D] / len_ref[...]


def _tc_sum_body(len_ref, x_ref, o_ref):
    b = pl.program_id(0)
    l = pl.program_id(1)
    len_b = len_ref[b]
    nlive = (len_b + RB_TC - 1) // RB_TC
    x = x_ref[0]
    rows = lax.broadcasted_iota(jnp.int32, (RB_TC, 1), 0) + l * RB_TC
    s = jnp.sum(jnp.where(rows < len_b, x, 0.0), axis=0)[None, None]

    @pl.when(l == 0)
    def _init():
        o_ref[...] = jnp.zeros_like(o_ref)

    @pl.when(l < nlive)
    def _acc():
        o_ref[...] = o_ref[...] + s


def _tc_sum(sentences, tc_lens):
    grid_spec = pltpu.PrefetchScalarGridSpec(
        num_scalar_prefetch=1,
        grid=(B, NL_TC),
        in_specs=[
            pl.BlockSpec(
                (1, RB_TC, D),
                lambda b, l, lens: (
                    b,
                    jnp.maximum(
                        jnp.minimum(l, (lens[b] + RB_TC - 1) // RB_TC - 1), 0
                    ),
                    0,
                ),
            ),
        ],
        out_specs=pl.BlockSpec((1, 1, D), lambda b, l, lens: (b, 0, 0)),
    )
    return pl.pallas_call(
        _tc_sum_body,
        grid_spec=grid_spec,
        out_shape=jax.ShapeDtypeStruct((B, 1, D), jnp.float32),
        compiler_params=pltpu.CompilerParams(
            dimension_semantics=("arbitrary", "arbitrary"),
        ),
    )(tc_lens, sentences)


def _combine_finish(p_ref, t_ref, len_ref, o_ref):
    s = jnp.sum(p_ref[...], axis=0)
    o_ref[...] = (s[:, :D] + t_ref[...]) / len_ref[...]


FTC_NUM = 5
FTC_DEN = 8


def _combined(sentences, sentence_lengths):
    tc_lens = (sentence_lengths * FTC_NUM // FTC_DEN) // RB * RB
    x = sentences.reshape(-1)
    t = _tc_sum(sentences, tc_lens)
    p = _make_sc_partial_sums()(x, sentence_lengths, tc_lens)
    return pl.pallas_call(
        _combine_finish,
        out_shape=jax.ShapeDtypeStruct((B, D), jnp.float32),
    )(
        p.reshape(NTILES, B, ACC_W),
        t.reshape(B, D),
        sentence_lengths.astype(jnp.float32).reshape(B, 1),
    )


def kernel(sentences, sentence_lengths):
    return _tc_native(sentences, sentence_lengths)
